# Initial kernel scaffold; baseline (speedup 1.0000x reference)
import jax
import jax.numpy as jnp
from jax.experimental import pallas as pl


def kernel(x, edge_index, edge_attr, batch, pat_idxs, params):
    raise NotImplementedError("write your pallas kernel here")


# speedup vs baseline: 2.8580x; 2.8580x over previous
import functools

import jax
import jax.numpy as jnp
from jax import lax
from jax.experimental import pallas as pl
from jax.experimental.pallas import tpu as pltpu
from jax.experimental.pallas import tpu_sc as plsc

_N, _E, _DF, _NH, _G, _NPAT = 20000, 64000, 1036, 128, 64, 16
_RATIO = 0.2
_INTERP = False


def _mm_body(x_ref, w_ref, u_ref, v_ref):
    y = jnp.dot(x_ref[...], w_ref[...], preferred_element_type=jnp.float32)
    u_ref[...] = y[:, :128]
    v_ref[...] = y[:, 128:]


def _mm(x, wt, bn):
    n, k = x.shape
    return pl.pallas_call(
        _mm_body,
        grid=(n // bn,),
        in_specs=[pl.BlockSpec((bn, k), lambda i: (i, 0)),
                  pl.BlockSpec((k, 256), lambda i: (0, 0))],
        out_specs=[pl.BlockSpec((bn, 128), lambda i: (i, 0)),
                   pl.BlockSpec((bn, 128), lambda i: (i, 0))],
        out_shape=[jax.ShapeDtypeStruct((n, 128), jnp.float32),
                   jax.ShapeDtypeStruct((n, 128), jnp.float32)],
        interpret=_INTERP,
    )(x, wt)


def _post_body(s_ref, c_ref, v_ref, vw_ref, bl_ref, x_ref, xm_ref):
    vwb = vw_ref[...]
    c = vwb * c_ref[...]
    agg = s_ref[...] / jnp.maximum(c, 1.0)
    xn = jnp.maximum(agg + bl_ref[...] + v_ref[...], 0.0)
    x_ref[...] = xn
    xm_ref[...] = xn * vwb


def _post(S, C1, v, vw, bl, bn=2000):
    return pl.pallas_call(
        _post_body,
        grid=(_N // bn,),
        in_specs=[pl.BlockSpec((bn, 128), lambda i: (i, 0)),
                  pl.BlockSpec((bn, 1), lambda i: (i, 0)),
                  pl.BlockSpec((bn, 128), lambda i: (i, 0)),
                  pl.BlockSpec((bn, 1), lambda i: (i, 0)),
                  pl.BlockSpec((1, 128), lambda i: (0, 0))],
        out_specs=[pl.BlockSpec((bn, 128), lambda i: (i, 0)),
                   pl.BlockSpec((bn, 128), lambda i: (i, 0))],
        out_shape=[jax.ShapeDtypeStruct((_N, 128), jnp.float32),
                   jax.ShapeDtypeStruct((_N, 128), jnp.float32)],
        interpret=_INTERP,
    )(S, C1, v, vw, bl.reshape(1, 128))


_NT = 16
_CH = 80
_EPT1 = _E // _NT
_NCH1 = _EPT1 // _CH
_EPT2 = _E // 2 // _NT
_HN = _N // 2
_AW = _HN + 8
_WPT = 624
_NR = 160

_sc_mesh = plsc.VectorSubcoreMesh(core_axis_name="c", subcore_axis_name="s")


def _fill_iota(idx_ref, base):
    lanes = lax.iota(jnp.int32, 16)
    for j in range(5):
        idx_ref[pl.ds(j * 16, 16)] = lanes + (base + j * 16)


def _acc_reduce(accc, idx_a, idx_b, shared):
    _fill_iota(idx_a, 0)
    _fill_iota(idx_b, 80)
    pltpu.sync_copy(accc.at[pl.ds(0, 80)], shared.at[idx_a], add=True)
    pltpu.sync_copy(accc.at[pl.ds(80, 80)], shared.at[idx_b], add=True)


def _seg_add16(accc, val_v, src_v, dst_v, i):
    s16 = src_v[pl.ds(i * 16, 16)]
    d16 = dst_v[pl.ds(i * 16, 16)]
    v16 = plsc.load_gather(val_v, [s16])
    row = lax.shift_right_logical(d16, 7)
    col = d16 & 127
    cur = plsc.load_gather(accc, [row, col])
    plsc.store_scatter(accc, [row, col], cur + v16)


@functools.partial(
    pl.kernel, mesh=_sc_mesh,
    out_type=jax.ShapeDtypeStruct((_N, 128), jnp.float32),
    scratch_types=[pltpu.VMEM((_CH,), jnp.int32),
                   pltpu.VMEM((_CH,), jnp.int32),
                   pltpu.VMEM((_CH,), jnp.int32),
                   pltpu.VMEM((_CH, 128), jnp.float32),
                   pltpu.VMEM_SHARED((_AW, 128), jnp.float32),
                   pltpu.SemaphoreType.DMA],
    compiler_params=pltpu.CompilerParams(needs_layout_passes=False),
)
def _edge_wide_sc(u, src, dst, zw, out_s,
                  src_v, dst_v, dstl_v, rows_v, acc_w, sem):
    cid = lax.axis_index("c")
    sid = lax.axis_index("s")
    r0 = sid * _WPT
    pltpu.sync_copy(zw.at[pl.ds(0, _WPT)], acc_w.at[pl.ds(r0, _WPT)])

    @pl.when(sid == 0)
    def _():
        pltpu.sync_copy(zw.at[pl.ds(0, 24)], acc_w.at[pl.ds(_NT * _WPT, 24)])

    plsc.subcore_barrier()

    base = cid * _HN

    def chunk(c, carry):
        off = pl.multiple_of(sid * _EPT1 + c * _CH, 8)
        pltpu.sync_copy(src.at[pl.ds(off, _CH)], src_v)
        pltpu.sync_copy(dst.at[pl.ds(off, _CH)], dst_v)
        for j in range(5):
            d16 = dst_v[pl.ds(j * 16, 16)]
            dl = d16 - base
            ok = (dl >= 0) & (dl < _HN)
            dstl_v[pl.ds(j * 16, 16)] = jnp.where(ok, dl, _HN)
        pltpu.async_copy(u.at[src_v], rows_v, sem).wait()
        pltpu.sync_copy(rows_v, acc_w.at[dstl_v], add=True)
        return carry

    lax.fori_loop(0, _NCH1, chunk, 0)
    plsc.subcore_barrier()
    pltpu.sync_copy(acc_w.at[pl.ds(r0, _WPT)],
                    out_s.at[pl.ds(base + r0, _WPT)])

    @pl.when(sid == 0)
    def _():
        pltpu.sync_copy(acc_w.at[pl.ds(_NT * _WPT, 16)],
                        out_s.at[pl.ds(base + _NT * _WPT, 16)])


@functools.partial(
    pl.kernel, mesh=_sc_mesh,
    out_type=[jax.ShapeDtypeStruct((_NR, 128), jnp.float32),
              jax.ShapeDtypeStruct((_NR, 128), jnp.float32)],
    scratch_types=[pltpu.VMEM((_EPT2,), jnp.int32),
                   pltpu.VMEM((_EPT2,), jnp.int32),
                   pltpu.VMEM((_NR, 128), jnp.float32),
                   pltpu.VMEM((_N,), jnp.float32),
                   pltpu.VMEM((_CH,), jnp.int32),
                   pltpu.VMEM((_CH,), jnp.int32),
                   pltpu.VMEM_SHARED((_NR, 128), jnp.float32),
                   pltpu.SemaphoreType.DMA],
    compiler_params=pltpu.CompilerParams(needs_layout_passes=False),
)
def _edge_scalar_sc(val, src, dst, zs, out0, out1,
                    src_v, dst_v, accc, val_v, idx_a, idx_b, acc_c, sem):
    cid = lax.axis_index("c")
    sid = lax.axis_index("s")

    @pl.when(sid == 0)
    def _():
        pltpu.sync_copy(zs, acc_c)

    pltpu.sync_copy(zs, accc)
    pltpu.sync_copy(val, val_v)
    eoff = pl.multiple_of(cid * (_E // 2) + sid * _EPT2, 8)
    pltpu.sync_copy(src.at[pl.ds(eoff, _EPT2)], src_v)
    pltpu.sync_copy(dst.at[pl.ds(eoff, _EPT2)], dst_v)
    plsc.subcore_barrier()

    def step(i, carry):
        _seg_add16(accc, val_v, src_v, dst_v, i)
        return carry

    lax.fori_loop(0, _EPT2 // 16, step, 0)
    _acc_reduce(accc, idx_a, idx_b, acc_c)
    plsc.subcore_barrier()

    @pl.when(sid == 0)
    def _():
        @pl.when(cid == 0)
        def _():
            pltpu.sync_copy(acc_c, out0)

        @pl.when(cid == 1)
        def _():
            pltpu.sync_copy(acc_c, out1)


def _edge_agg(u, src, dst):
    zw = jnp.zeros((_WPT, 128), jnp.float32)
    return _edge_wide_sc(u, src, dst, zw)


def _edge_counts(vw, src, dst):
    zs = jnp.zeros((_NR, 128), jnp.float32)
    c0, c1 = _edge_scalar_sc(vw[:, 0], src, dst, zs)
    return (c0 + c1).reshape(-1)[:_N, None]


def _topk(score, batch, valid):
    counts = jnp.bincount(batch, length=_G)
    vcounts = jax.ops.segment_sum(valid.astype(jnp.int32), batch,
                                  num_segments=_G)
    k = jnp.ceil(_RATIO * vcounts.astype(jnp.float32)).astype(jnp.int32)
    s = jnp.where(valid, score, -jnp.inf)
    order = jnp.lexsort((-s, batch))
    starts = (jnp.cumsum(counts) - counts).astype(jnp.int32)
    rank = jnp.zeros((_N,), jnp.int32).at[order].set(
        jnp.arange(_N, dtype=jnp.int32) - starts[batch[order]])
    return (rank < k[batch]) & valid


def kernel(x, edge_index, edge_attr, batch, pat_idxs, params):
    p = params
    src = edge_index[0]
    dst = edge_index[1]
    colmax = jnp.max(x[:, :12], axis=0)
    scale = jnp.ones((_DF,), jnp.float32).at[:12].set(1.0 / colmax)
    vw = jnp.ones((_N, 1), jnp.float32)
    xin = x
    xs_acc = None
    for i in range(3):
        cin = _DF if i == 0 else _NH
        wt = jnp.concatenate([p['cWl%d' % i], p['cWr%d' % i]], axis=0).T
        if i == 0:
            wt = wt * scale[:, None]
        u, v = _mm(xin, wt, 800 if cin > 512 else 2000)
        S = _edge_agg(u, src, dst)
        C1 = _edge_counts(vw, src, dst)
        xn, xnm = _post(S, C1, v, vw, p['cbl%d' % i])
        Rv = _edge_agg(xnm, src, dst)
        score = jnp.tanh(Rv @ p['pWrel%d' % i].T + p['pbrel%d' % i]
                         + xn @ p['pWroot%d' % i].T)[:, 0]
        valid = vw[:, 0] > 0
        nmask = _topk(score, batch, valid)
        vw_new = nmask.astype(jnp.float32)
        xm = xn * (score * vw_new)[:, None]
        gm = jax.ops.segment_max(jnp.where(nmask[:, None], xm, -jnp.inf),
                                 batch, num_segments=_G)
        sgs = jax.ops.segment_sum(xm, batch, num_segments=_G)
        cg = jax.ops.segment_sum(vw_new, batch, num_segments=_G)
        ga = sgs / jnp.maximum(cg, 1.0)[:, None]
        xsl = jnp.concatenate([gm, ga], axis=1)
        xs_acc = xsl if xs_acc is None else xs_acc + xsl
        xin = xm
        vw = vw_new[:, None]
    cnt_p = jax.ops.segment_sum(jnp.ones((_G,), jnp.float32), pat_idxs,
                                num_segments=_NPAT)
    xp = (jax.ops.segment_sum(xs_acc, pat_idxs, num_segments=_NPAT)
          / jnp.maximum(cnt_p, 1.0)[:, None])
    h = jnp.maximum(xp @ p['l1W'].T + p['l1b'], 0.0)
    feats = jnp.maximum(h @ p['l2W'].T + p['l2b'], 0.0)
    g = feats @ p['gW'].T + p['gb']
    grade = jax.nn.log_softmax(g, axis=1)
    hazard = jax.nn.sigmoid(feats @ p['hW'].T + p['hb']) * 6.0 - 3.0
    return feats, grade, hazard


# speedup vs baseline: 3.4911x; 1.2215x over previous
import functools

import jax
import jax.numpy as jnp
from jax import lax
from jax.experimental import pallas as pl
from jax.experimental.pallas import tpu as pltpu
from jax.experimental.pallas import tpu_sc as plsc

_N, _E, _DF, _NH, _G, _NPAT = 20000, 64000, 1036, 128, 64, 16
_RATIO = 0.2
_INTERP = False


def _mm_body(x_ref, w_ref, u_ref, v_ref):
    y = jnp.dot(x_ref[...], w_ref[...], preferred_element_type=jnp.float32)
    u_ref[...] = y[:, :128]
    v_ref[...] = y[:, 128:]


def _mm(x, wt, bn):
    n, k = x.shape
    return pl.pallas_call(
        _mm_body,
        grid=(n // bn,),
        in_specs=[pl.BlockSpec((bn, k), lambda i: (i, 0)),
                  pl.BlockSpec((k, 256), lambda i: (0, 0))],
        out_specs=[pl.BlockSpec((bn, 128), lambda i: (i, 0)),
                   pl.BlockSpec((bn, 128), lambda i: (i, 0))],
        out_shape=[jax.ShapeDtypeStruct((n, 128), jnp.float32),
                   jax.ShapeDtypeStruct((n, 128), jnp.float32)],
        interpret=_INTERP,
    )(x, wt)


def _post_body(s_ref, c_ref, v_ref, vw_ref, bl_ref, x_ref, xm_ref):
    vwb = vw_ref[...]
    c = vwb * c_ref[...]
    agg = s_ref[...] / jnp.maximum(c, 1.0)
    xn = jnp.maximum(agg + bl_ref[...] + v_ref[...], 0.0)
    x_ref[...] = xn
    xm_ref[...] = xn * vwb


def _post(S, C1, v, vw, bl, bn=2000):
    return pl.pallas_call(
        _post_body,
        grid=(_N // bn,),
        in_specs=[pl.BlockSpec((bn, 128), lambda i: (i, 0)),
                  pl.BlockSpec((bn, 1), lambda i: (i, 0)),
                  pl.BlockSpec((bn, 128), lambda i: (i, 0)),
                  pl.BlockSpec((bn, 1), lambda i: (i, 0)),
                  pl.BlockSpec((1, 128), lambda i: (0, 0))],
        out_specs=[pl.BlockSpec((bn, 128), lambda i: (i, 0)),
                   pl.BlockSpec((bn, 128), lambda i: (i, 0))],
        out_shape=[jax.ShapeDtypeStruct((_N, 128), jnp.float32),
                   jax.ShapeDtypeStruct((_N, 128), jnp.float32)],
        interpret=_INTERP,
    )(S, C1, v, vw, bl.reshape(1, 128))


_NT = 16
_CH = 80
_EPT1 = _E // _NT
_NCH1 = _EPT1 // _CH
_EPT2 = _E // 2 // _NT
_HN = _N // 2
_AW = _HN + 8
_WPT = 624
_NR = 160

_sc_mesh = plsc.VectorSubcoreMesh(core_axis_name="c", subcore_axis_name="s")


def _fill_iota(idx_ref, base):
    lanes = lax.iota(jnp.int32, 16)
    for j in range(5):
        idx_ref[pl.ds(j * 16, 16)] = lanes + (base + j * 16)


def _acc_reduce(accc, idx_a, idx_b, shared):
    _fill_iota(idx_a, 0)
    _fill_iota(idx_b, 80)
    pltpu.sync_copy(accc.at[pl.ds(0, 80)], shared.at[idx_a], add=True)
    pltpu.sync_copy(accc.at[pl.ds(80, 80)], shared.at[idx_b], add=True)


def _seg_add16(accc, val_v, src_v, dst_v, i):
    s16 = src_v[pl.ds(i * 16, 16)]
    d16 = dst_v[pl.ds(i * 16, 16)]
    v16 = plsc.load_gather(val_v, [s16])
    row = lax.shift_right_logical(d16, 7)
    col = d16 & 127
    cur = plsc.load_gather(accc, [row, col])
    plsc.store_scatter(accc, [row, col], cur + v16)


@functools.partial(
    pl.kernel, mesh=_sc_mesh,
    out_type=jax.ShapeDtypeStruct((_N, 128), jnp.float32),
    scratch_types=[pltpu.VMEM((_EPT1,), jnp.int32),
                   pltpu.VMEM((_EPT1,), jnp.int32),
                   pltpu.VMEM((4, _CH), jnp.int32),
                   pltpu.VMEM((4, _CH, 128), jnp.float32),
                   pltpu.VMEM_SHARED((_AW, 128), jnp.float32),
                   pltpu.SemaphoreType.DMA,
                   pltpu.SemaphoreType.DMA],
    compiler_params=pltpu.CompilerParams(needs_layout_passes=False),
)
def _edge_wide_sc(u, src, dst, zw, out_s,
                  src_v, dst_v, dstl_v, rows_v, acc_w, sem_g, sem_s):
    cid = lax.axis_index("c")
    sid = lax.axis_index("s")
    r0 = sid * _WPT
    pltpu.sync_copy(zw.at[pl.ds(0, _WPT)], acc_w.at[pl.ds(r0, _WPT)])

    @pl.when(sid == 0)
    def _():
        pltpu.sync_copy(zw.at[pl.ds(0, 24)], acc_w.at[pl.ds(_NT * _WPT, 24)])

    eoff = pl.multiple_of(sid * _EPT1, 8)
    pltpu.sync_copy(src.at[pl.ds(eoff, _EPT1)], src_v)
    pltpu.sync_copy(dst.at[pl.ds(eoff, _EPT1)], dst_v)
    plsc.subcore_barrier()

    base = cid * _HN

    def fire_gather(g):
        pltpu.async_copy(u.at[src_v.at[pl.ds(g * _CH, _CH)]],
                         rows_v.at[g % 4], sem_g)

    def drain(sem):
        pltpu.make_async_copy(u.at[pl.ds(0, _CH)], rows_v.at[0], sem).wait()

    fire_gather(0)
    fire_gather(1)

    def chunk(g, carry):
        @pl.when(g >= 2)
        def _():
            drain(sem_s)

        @pl.when(g < _NCH1 - 2)
        def _():
            fire_gather(g + 2)

        drain(sem_g)
        b = g % 4
        for j in range(5):
            d16 = dst_v[pl.ds(g * _CH + j * 16, 16)]
            dl = d16 - base
            ok = (dl >= 0) & (dl < _HN)
            dstl_v[b, pl.ds(j * 16, 16)] = jnp.where(ok, dl, _HN)
        pltpu.async_copy(rows_v.at[b], acc_w.at[dstl_v.at[b]], sem_s,
                         add=True)
        return carry

    lax.fori_loop(0, _NCH1, chunk, 0)
    drain(sem_s)
    drain(sem_s)
    plsc.subcore_barrier()
    pltpu.sync_copy(acc_w.at[pl.ds(r0, _WPT)],
                    out_s.at[pl.ds(base + r0, _WPT)])

    @pl.when(sid == 0)
    def _():
        pltpu.sync_copy(acc_w.at[pl.ds(_NT * _WPT, 16)],
                        out_s.at[pl.ds(base + _NT * _WPT, 16)])


@functools.partial(
    pl.kernel, mesh=_sc_mesh,
    out_type=[jax.ShapeDtypeStruct((_NR, 128), jnp.float32),
              jax.ShapeDtypeStruct((_NR, 128), jnp.float32)],
    scratch_types=[pltpu.VMEM((_EPT2,), jnp.int32),
                   pltpu.VMEM((_EPT2,), jnp.int32),
                   pltpu.VMEM((_NR, 128), jnp.float32),
                   pltpu.VMEM((_N,), jnp.float32),
                   pltpu.VMEM((_CH,), jnp.int32),
                   pltpu.VMEM((_CH,), jnp.int32),
                   pltpu.VMEM_SHARED((_NR, 128), jnp.float32),
                   pltpu.SemaphoreType.DMA],
    compiler_params=pltpu.CompilerParams(needs_layout_passes=False),
)
def _edge_scalar_sc(val, src, dst, zs, out0, out1,
                    src_v, dst_v, accc, val_v, idx_a, idx_b, acc_c, sem):
    cid = lax.axis_index("c")
    sid = lax.axis_index("s")

    @pl.when(sid == 0)
    def _():
        pltpu.sync_copy(zs, acc_c)

    pltpu.sync_copy(zs, accc)
    pltpu.sync_copy(val, val_v)
    eoff = pl.multiple_of(cid * (_E // 2) + sid * _EPT2, 8)
    pltpu.sync_copy(src.at[pl.ds(eoff, _EPT2)], src_v)
    pltpu.sync_copy(dst.at[pl.ds(eoff, _EPT2)], dst_v)
    plsc.subcore_barrier()

    def step(i, carry):
        _seg_add16(accc, val_v, src_v, dst_v, i)
        return carry

    lax.fori_loop(0, _EPT2 // 16, step, 0)
    _acc_reduce(accc, idx_a, idx_b, acc_c)
    plsc.subcore_barrier()

    @pl.when(sid == 0)
    def _():
        @pl.when(cid == 0)
        def _():
            pltpu.sync_copy(acc_c, out0)

        @pl.when(cid == 1)
        def _():
            pltpu.sync_copy(acc_c, out1)


def _edge_agg(u, src, dst):
    zw = jnp.zeros((_WPT, 128), jnp.float32)
    return _edge_wide_sc(u, src, dst, zw)


def _edge_counts(vw, src, dst):
    zs = jnp.zeros((_NR, 128), jnp.float32)
    c0, c1 = _edge_scalar_sc(vw[:, 0], src, dst, zs)
    return (c0 + c1).reshape(-1)[:_N, None]


def _topk(score, batch, valid):
    counts = jnp.bincount(batch, length=_G)
    vcounts = jax.ops.segment_sum(valid.astype(jnp.int32), batch,
                                  num_segments=_G)
    k = jnp.ceil(_RATIO * vcounts.astype(jnp.float32)).astype(jnp.int32)
    s = jnp.where(valid, score, -jnp.inf)
    order = jnp.lexsort((-s, batch))
    starts = (jnp.cumsum(counts) - counts).astype(jnp.int32)
    rank = jnp.zeros((_N,), jnp.int32).at[order].set(
        jnp.arange(_N, dtype=jnp.int32) - starts[batch[order]])
    return (rank < k[batch]) & valid


def kernel(x, edge_index, edge_attr, batch, pat_idxs, params):
    p = params
    src = edge_index[0]
    dst = edge_index[1]
    colmax = jnp.max(x[:, :12], axis=0)
    scale = jnp.ones((_DF,), jnp.float32).at[:12].set(1.0 / colmax)
    vw = jnp.ones((_N, 1), jnp.float32)
    xin = x
    xs_acc = None
    for i in range(3):
        cin = _DF if i == 0 else _NH
        wt = jnp.concatenate([p['cWl%d' % i], p['cWr%d' % i]], axis=0).T
        if i == 0:
            wt = wt * scale[:, None]
        u, v = _mm(xin, wt, 800 if cin > 512 else 2000)
        S = _edge_agg(u, src, dst)
        C1 = _edge_counts(vw, src, dst)
        xn, xnm = _post(S, C1, v, vw, p['cbl%d' % i])
        Rv = _edge_agg(xnm, src, dst)
        score = jnp.tanh(Rv @ p['pWrel%d' % i].T + p['pbrel%d' % i]
                         + xn @ p['pWroot%d' % i].T)[:, 0]
        valid = vw[:, 0] > 0
        nmask = _topk(score, batch, valid)
        vw_new = nmask.astype(jnp.float32)
        xm = xn * (score * vw_new)[:, None]
        gm = jax.ops.segment_max(jnp.where(nmask[:, None], xm, -jnp.inf),
                                 batch, num_segments=_G)
        sgs = jax.ops.segment_sum(xm, batch, num_segments=_G)
        cg = jax.ops.segment_sum(vw_new, batch, num_segments=_G)
        ga = sgs / jnp.maximum(cg, 1.0)[:, None]
        xsl = jnp.concatenate([gm, ga], axis=1)
        xs_acc = xsl if xs_acc is None else xs_acc + xsl
        xin = xm
        vw = vw_new[:, None]
    cnt_p = jax.ops.segment_sum(jnp.ones((_G,), jnp.float32), pat_idxs,
                                num_segments=_NPAT)
    xp = (jax.ops.segment_sum(xs_acc, pat_idxs, num_segments=_NPAT)
          / jnp.maximum(cnt_p, 1.0)[:, None])
    h = jnp.maximum(xp @ p['l1W'].T + p['l1b'], 0.0)
    feats = jnp.maximum(h @ p['l2W'].T + p['l2b'], 0.0)
    g = feats @ p['gW'].T + p['gb']
    grade = jax.nn.log_softmax(g, axis=1)
    hazard = jax.nn.sigmoid(feats @ p['hW'].T + p['hb']) * 6.0 - 3.0
    return feats, grade, hazard


# speedup vs baseline: 3.5380x; 1.0134x over previous
import functools

import jax
import jax.numpy as jnp
from jax import lax
from jax.experimental import pallas as pl
from jax.experimental.pallas import tpu as pltpu
from jax.experimental.pallas import tpu_sc as plsc

_N, _E, _DF, _NH, _G, _NPAT = 20000, 64000, 1036, 128, 64, 16
_RATIO = 0.2


def _mm_body(x_ref, w_ref, u_ref, v_ref):
    y = jnp.dot(x_ref[...], w_ref[...], preferred_element_type=jnp.float32)
    u_ref[...] = y[:, :128]
    v_ref[...] = y[:, 128:]


def _mm(x, wt, bn):
    n, k = x.shape
    return pl.pallas_call(
        _mm_body,
        grid=(n // bn,),
        in_specs=[pl.BlockSpec((bn, k), lambda i: (i, 0)),
                  pl.BlockSpec((k, 256), lambda i: (0, 0))],
        out_specs=[pl.BlockSpec((bn, 128), lambda i: (i, 0)),
                   pl.BlockSpec((bn, 128), lambda i: (i, 0))],
        out_shape=[jax.ShapeDtypeStruct((n, 128), jnp.float32),
                   jax.ShapeDtypeStruct((n, 128), jnp.float32)],
    )(x, wt)


def _post_body(s_ref, c_ref, v_ref, vw_ref, bl_ref, x_ref, xm_ref):
    vwb = vw_ref[...]
    c = vwb * c_ref[...]
    agg = s_ref[...] / jnp.maximum(c, 1.0)
    xn = jnp.maximum(agg + bl_ref[...] + v_ref[...], 0.0)
    x_ref[...] = xn
    xm_ref[...] = xn * vwb


def _post2_body(s_ref, c_ref, x_ref, vw_ref, wl_ref, wr_ref, bl_ref,
                xo_ref, xm_ref):
    vwb = vw_ref[...]
    c = vwb * c_ref[...]
    agg = s_ref[...] / jnp.maximum(c, 1.0)
    y = (jnp.dot(agg, wl_ref[...], preferred_element_type=jnp.float32)
         + bl_ref[...]
         + jnp.dot(x_ref[...], wr_ref[...],
                   preferred_element_type=jnp.float32))
    xn = jnp.maximum(y, 0.0)
    xo_ref[...] = xn
    xm_ref[...] = xn * vwb


def _post2(S, C1, x, vw, wlT, wrT, bl, bn=2000):
    return pl.pallas_call(
        _post2_body,
        grid=(_N // bn,),
        in_specs=[pl.BlockSpec((bn, 128), lambda i: (i, 0)),
                  pl.BlockSpec((bn, 1), lambda i: (i, 0)),
                  pl.BlockSpec((bn, 128), lambda i: (i, 0)),
                  pl.BlockSpec((bn, 1), lambda i: (i, 0)),
                  pl.BlockSpec((128, 128), lambda i: (0, 0)),
                  pl.BlockSpec((128, 128), lambda i: (0, 0)),
                  pl.BlockSpec((1, 128), lambda i: (0, 0))],
        out_specs=[pl.BlockSpec((bn, 128), lambda i: (i, 0)),
                   pl.BlockSpec((bn, 128), lambda i: (i, 0))],
        out_shape=[jax.ShapeDtypeStruct((_N, 128), jnp.float32),
                   jax.ShapeDtypeStruct((_N, 128), jnp.float32)],
    )(S, C1, x, vw, wlT, wrT, bl.reshape(1, 128))


def _post(S, C1, v, vw, bl, bn=2000):
    return pl.pallas_call(
        _post_body,
        grid=(_N // bn,),
        in_specs=[pl.BlockSpec((bn, 128), lambda i: (i, 0)),
                  pl.BlockSpec((bn, 1), lambda i: (i, 0)),
                  pl.BlockSpec((bn, 128), lambda i: (i, 0)),
                  pl.BlockSpec((bn, 1), lambda i: (i, 0)),
                  pl.BlockSpec((1, 128), lambda i: (0, 0))],
        out_specs=[pl.BlockSpec((bn, 128), lambda i: (i, 0)),
                   pl.BlockSpec((bn, 128), lambda i: (i, 0))],
        out_shape=[jax.ShapeDtypeStruct((_N, 128), jnp.float32),
                   jax.ShapeDtypeStruct((_N, 128), jnp.float32)],
    )(S, C1, v, vw, bl.reshape(1, 128))


_NT = 16
_CH = 80
_EPT1 = _E // _NT
_NCH1 = _EPT1 // _CH
_EPT2 = _E // 2 // _NT
_HN = _N // 2
_AW = _HN + 8
_WPT = 624
_NR = 160

_sc_mesh = plsc.VectorSubcoreMesh(core_axis_name="c", subcore_axis_name="s")


def _fill_iota(idx_ref, base):
    lanes = lax.iota(jnp.int32, 16)
    for j in range(5):
        idx_ref[pl.ds(j * 16, 16)] = lanes + (base + j * 16)


def _acc_reduce(accc, idx_a, idx_b, shared):
    _fill_iota(idx_a, 0)
    _fill_iota(idx_b, 80)
    pltpu.sync_copy(accc.at[pl.ds(0, 80)], shared.at[idx_a], add=True)
    pltpu.sync_copy(accc.at[pl.ds(80, 80)], shared.at[idx_b], add=True)


def _seg_add16(accc, val_v, src_v, dst_v, i):
    s16 = src_v[pl.ds(i * 16, 16)]
    d16 = dst_v[pl.ds(i * 16, 16)]
    v16 = plsc.load_gather(val_v, [s16])
    row = lax.shift_right_logical(d16, 7)
    col = d16 & 127
    cur = plsc.load_gather(accc, [row, col])
    plsc.store_scatter(accc, [row, col], cur + v16)


@functools.partial(
    pl.kernel, mesh=_sc_mesh,
    out_type=jax.ShapeDtypeStruct((_N, 128), jnp.float32),
    scratch_types=[pltpu.VMEM((_EPT1,), jnp.int32),
                   pltpu.VMEM((_EPT1,), jnp.int32),
                   pltpu.VMEM((4, _CH), jnp.int32),
                   pltpu.VMEM((4, _CH, 128), jnp.float32),
                   pltpu.VMEM_SHARED((_AW, 128), jnp.float32),
                   pltpu.SemaphoreType.DMA,
                   pltpu.SemaphoreType.DMA],
    compiler_params=pltpu.CompilerParams(needs_layout_passes=False),
)
def _edge_wide_sc(u, src, dst, zw, out_s,
                  src_v, dst_v, dstl_v, rows_v, acc_w, sem_g, sem_s):
    cid = lax.axis_index("c")
    sid = lax.axis_index("s")
    r0 = sid * _WPT
    pltpu.sync_copy(zw.at[pl.ds(0, _WPT)], acc_w.at[pl.ds(r0, _WPT)])

    @pl.when(sid == 0)
    def _():
        pltpu.sync_copy(zw.at[pl.ds(0, 24)], acc_w.at[pl.ds(_NT * _WPT, 24)])

    eoff = pl.multiple_of(sid * _EPT1, 8)
    pltpu.sync_copy(src.at[pl.ds(eoff, _EPT1)], src_v)
    pltpu.sync_copy(dst.at[pl.ds(eoff, _EPT1)], dst_v)
    plsc.subcore_barrier()

    base = cid * _HN

    def fire_gather(g):
        pltpu.async_copy(u.at[src_v.at[pl.ds(g * _CH, _CH)]],
                         rows_v.at[g % 4], sem_g)

    def drain(sem):
        pltpu.make_async_copy(u.at[pl.ds(0, _CH)], rows_v.at[0], sem).wait()

    fire_gather(0)
    fire_gather(1)

    def chunk(g, carry):
        @pl.when(g >= 2)
        def _():
            drain(sem_s)

        @pl.when(g < _NCH1 - 2)
        def _():
            fire_gather(g + 2)

        drain(sem_g)
        b = g % 4
        for j in range(5):
            d16 = dst_v[pl.ds(g * _CH + j * 16, 16)]
            dl = d16 - base
            ok = (dl >= 0) & (dl < _HN)
            dstl_v[b, pl.ds(j * 16, 16)] = jnp.where(ok, dl, _HN)
        pltpu.async_copy(rows_v.at[b], acc_w.at[dstl_v.at[b]], sem_s,
                         add=True)
        return carry

    lax.fori_loop(0, _NCH1, chunk, 0)
    drain(sem_s)
    drain(sem_s)
    plsc.subcore_barrier()
    pltpu.sync_copy(acc_w.at[pl.ds(r0, _WPT)],
                    out_s.at[pl.ds(base + r0, _WPT)])

    @pl.when(sid == 0)
    def _():
        pltpu.sync_copy(acc_w.at[pl.ds(_NT * _WPT, 16)],
                        out_s.at[pl.ds(base + _NT * _WPT, 16)])


@functools.partial(
    pl.kernel, mesh=_sc_mesh,
    out_type=[jax.ShapeDtypeStruct((_NR, 128), jnp.float32),
              jax.ShapeDtypeStruct((_NR, 128), jnp.float32)],
    scratch_types=[pltpu.VMEM((_EPT2,), jnp.int32),
                   pltpu.VMEM((_EPT2,), jnp.int32),
                   pltpu.VMEM((_NR, 128), jnp.float32),
                   pltpu.VMEM((_N,), jnp.float32),
                   pltpu.VMEM((_CH,), jnp.int32),
                   pltpu.VMEM((_CH,), jnp.int32),
                   pltpu.VMEM_SHARED((_NR, 128), jnp.float32),
                   pltpu.SemaphoreType.DMA],
    compiler_params=pltpu.CompilerParams(needs_layout_passes=False),
)
def _edge_scalar_sc(val, src, dst, zs, out0, out1,
                    src_v, dst_v, accc, val_v, idx_a, idx_b, acc_c, sem):
    cid = lax.axis_index("c")
    sid = lax.axis_index("s")

    @pl.when(sid == 0)
    def _():
        pltpu.sync_copy(zs, acc_c)

    pltpu.sync_copy(zs, accc)
    pltpu.sync_copy(val, val_v)
    eoff = pl.multiple_of(cid * (_E // 2) + sid * _EPT2, 8)
    pltpu.sync_copy(src.at[pl.ds(eoff, _EPT2)], src_v)
    pltpu.sync_copy(dst.at[pl.ds(eoff, _EPT2)], dst_v)
    plsc.subcore_barrier()

    def step(i, carry):
        _seg_add16(accc, val_v, src_v, dst_v, i)
        return carry

    lax.fori_loop(0, _EPT2 // 16, step, 0)
    _acc_reduce(accc, idx_a, idx_b, acc_c)
    plsc.subcore_barrier()

    @pl.when(sid == 0)
    def _():
        @pl.when(cid == 0)
        def _():
            pltpu.sync_copy(acc_c, out0)

        @pl.when(cid == 1)
        def _():
            pltpu.sync_copy(acc_c, out1)


def _edge_agg(u, src, dst):
    zw = jnp.zeros((_WPT, 128), jnp.float32)
    return _edge_wide_sc(u, src, dst, zw)


def _edge_counts(vw, src, dst):
    zs = jnp.zeros((_NR, 128), jnp.float32)
    c0, c1 = _edge_scalar_sc(vw[:, 0], src, dst, zs)
    return (c0 + c1).reshape(-1)[:_N, None]


def _topk(score, batch, valid):
    counts = jnp.bincount(batch, length=_G)
    vcounts = jax.ops.segment_sum(valid.astype(jnp.int32), batch,
                                  num_segments=_G)
    k = jnp.ceil(_RATIO * vcounts.astype(jnp.float32)).astype(jnp.int32)
    s = jnp.where(valid, score, -jnp.inf)
    order = jnp.lexsort((-s, batch))
    starts = (jnp.cumsum(counts) - counts).astype(jnp.int32)
    rank = jnp.zeros((_N,), jnp.int32).at[order].set(
        jnp.arange(_N, dtype=jnp.int32) - starts[batch[order]])
    return (rank < k[batch]) & valid


def kernel(x, edge_index, edge_attr, batch, pat_idxs, params):
    p = params
    src = edge_index[0]
    dst = edge_index[1]
    colmax = jnp.max(x[:, :12], axis=0)
    scale = jnp.ones((_DF,), jnp.float32).at[:12].set(1.0 / colmax)
    vw = jnp.ones((_N, 1), jnp.float32)
    xin = x
    xs_acc = None
    for i in range(3):
        if i == 0:
            wt = jnp.concatenate([p['cWl0'], p['cWr0']], axis=0).T
            wt = wt * scale[:, None]
        if i == 0:
            u, v = _mm(xin, wt, 800)
            S = _edge_agg(u, src, dst)
            C1 = _edge_counts(vw, src, dst)
            xn, xnm = _post(S, C1, v, vw, p['cbl%d' % i])
        else:
            S = _edge_agg(xin, src, dst)
            C1 = _edge_counts(vw, src, dst)
            xn, xnm = _post2(S, C1, xin, vw, p['cWl%d' % i].T,
                             p['cWr%d' % i].T, p['cbl%d' % i])
        Rv = _edge_agg(xnm, src, dst)
        score = jnp.tanh(Rv @ p['pWrel%d' % i].T + p['pbrel%d' % i]
                         + xn @ p['pWroot%d' % i].T)[:, 0]
        valid = vw[:, 0] > 0
        nmask = _topk(score, batch, valid)
        vw_new = nmask.astype(jnp.float32)
        xm = xn * (score * vw_new)[:, None]
        gm = jax.ops.segment_max(jnp.where(nmask[:, None], xm, -jnp.inf),
                                 batch, num_segments=_G)
        sgs = jax.ops.segment_sum(xm, batch, num_segments=_G)
        cg = jax.ops.segment_sum(vw_new, batch, num_segments=_G)
        ga = sgs / jnp.maximum(cg, 1.0)[:, None]
        xsl = jnp.concatenate([gm, ga], axis=1)
        xs_acc = xsl if xs_acc is None else xs_acc + xsl
        xin = xm
        vw = vw_new[:, None]
    cnt_p = jax.ops.segment_sum(jnp.ones((_G,), jnp.float32), pat_idxs,
                                num_segments=_NPAT)
    xp = (jax.ops.segment_sum(xs_acc, pat_idxs, num_segments=_NPAT)
          / jnp.maximum(cnt_p, 1.0)[:, None])
    h = jnp.maximum(xp @ p['l1W'].T + p['l1b'], 0.0)
    feats = jnp.maximum(h @ p['l2W'].T + p['l2b'], 0.0)
    g = feats @ p['gW'].T + p['gb']
    grade = jax.nn.log_softmax(g, axis=1)
    hazard = jax.nn.sigmoid(feats @ p['hW'].T + p['hb']) * 6.0 - 3.0
    return feats, grade, hazard
